# Initial kernel scaffold; baseline (speedup 1.0000x reference)
import jax
import jax.numpy as jnp
from jax.experimental import pallas as pl


def kernel(features, edge_index, W, b):
    raise NotImplementedError("write your pallas kernel here")


# speedup vs baseline: 4.6926x; 4.6926x over previous
import functools

import jax
import jax.numpy as jnp
from jax import lax
from jax.experimental import pallas as pl
from jax.experimental.pallas import tpu as pltpu
from jax.experimental.pallas import tpu_sc as plsc

_NC = 2
_NS = 16
_NW = _NC * _NS
_L = 16


def _sc_mesh():
    return plsc.VectorSubcoreMesh(core_axis_name="c", subcore_axis_name="s")


@functools.lru_cache(maxsize=None)
def _make_deg_kernel(E: int, NP: int):
    EPT = E // _NW
    C = 80
    n_chunks = EPT // C
    assert n_chunks * C == EPT
    RPT = NP // _NS

    @functools.partial(
        pl.kernel,
        out_type=jax.ShapeDtypeStruct((_NC, 2, NP), jnp.float32),
        mesh=_sc_mesh(),
        scratch_types=[
            pltpu.VMEM((C,), jnp.int32),
            pltpu.VMEM((C,), jnp.int32),
            pltpu.VMEM((C,), jnp.float32),
            pltpu.VMEM((RPT,), jnp.float32),
            pltpu.VMEM_SHARED((NP,), jnp.float32),
            pltpu.VMEM_SHARED((NP,), jnp.float32),
        ],
    )
    def deg_kernel(src_hbm, dst_hbm, out_hbm, src_v, dst_v, ones_v, zero_v,
                   dout_sh, din_sh):
        c = lax.axis_index("c")
        s = lax.axis_index("s")
        wid = c * _NS + s

        for j in range(C // _L):
            ones_v[pl.ds(j * _L, _L)] = jnp.ones((_L,), jnp.float32)

        def zfill(i, carry):
            zero_v[pl.ds(i * _L, _L)] = jnp.zeros((_L,), jnp.float32)
            return carry
        lax.fori_loop(0, RPT // _L, zfill, 0)

        off = s * RPT
        pltpu.sync_copy(zero_v, dout_sh.at[pl.ds(off, RPT)])
        pltpu.sync_copy(zero_v, din_sh.at[pl.ds(off, RPT)])
        plsc.subcore_barrier()

        base = wid * EPT

        def body(j, carry):
            o = base + j * C
            pltpu.sync_copy(src_hbm.at[pl.ds(o, C)], src_v)
            pltpu.sync_copy(dst_hbm.at[pl.ds(o, C)], dst_v)
            pltpu.sync_copy(ones_v, dout_sh.at[src_v], add=True)
            pltpu.sync_copy(ones_v, din_sh.at[dst_v], add=True)
            return carry
        lax.fori_loop(0, n_chunks, body, 0)

        plsc.subcore_barrier()
        pltpu.sync_copy(dout_sh.at[pl.ds(off, RPT)],
                        out_hbm.at[c, 0, pl.ds(off, RPT)])
        pltpu.sync_copy(din_sh.at[pl.ds(off, RPT)],
                        out_hbm.at[c, 1, pl.ds(off, RPT)])

    return deg_kernel


@functools.lru_cache(maxsize=None)
def _make_agg_kernel(E: int, NP: int, D: int):
    EPT = E // _NW
    C = 80
    n_chunks = EPT // C
    assert n_chunks * C == EPT
    RPT = NP // _NS
    ZR = 64

    @functools.partial(
        pl.kernel,
        out_type=jax.ShapeDtypeStruct((_NC, NP, D), jnp.float32),
        mesh=_sc_mesh(),
        scratch_types=[
            pltpu.VMEM((C,), jnp.int32),
            pltpu.VMEM((C,), jnp.int32),
            pltpu.VMEM((C, D), jnp.float32),
            pltpu.VMEM((ZR, D), jnp.float32),
            pltpu.VMEM_SHARED((NP, D), jnp.float32),
            pltpu.SemaphoreType.DMA,
        ],
    )
    def agg_kernel(h_hbm, src_hbm, dst_hbm, out_hbm, src_v, dst_v, rows_v,
                   zbuf, agg_sh, sem):
        c = lax.axis_index("c")
        s = lax.axis_index("s")
        wid = c * _NS + s

        def zfill(i, carry):
            for j in range(D // _L):
                zbuf[i, pl.ds(j * _L, _L)] = jnp.zeros((_L,), jnp.float32)
            return carry
        lax.fori_loop(0, ZR, zfill, 0)

        def zcopy(k, carry):
            pltpu.sync_copy(zbuf, agg_sh.at[pl.ds(s * RPT + k * ZR, ZR), :])
            return carry
        lax.fori_loop(0, RPT // ZR, zcopy, 0)
        plsc.subcore_barrier()

        base = wid * EPT

        def body(j, carry):
            o = base + j * C
            pltpu.sync_copy(src_hbm.at[pl.ds(o, C)], src_v)
            pltpu.sync_copy(dst_hbm.at[pl.ds(o, C)], dst_v)
            pltpu.async_copy(h_hbm.at[src_v], rows_v, sem).wait()
            pltpu.sync_copy(rows_v, agg_sh.at[dst_v], add=True)
            return carry
        lax.fori_loop(0, n_chunks, body, 0)

        plsc.subcore_barrier()
        pltpu.sync_copy(agg_sh.at[pl.ds(s * RPT, RPT), :],
                        out_hbm.at[c, pl.ds(s * RPT, RPT), :])

    return agg_kernel


def _h_body(f_ref, d_ref, o_ref):
    d = d_ref[0] + d_ref[1]
    norm = lax.rsqrt(jnp.maximum(d, 1.0))
    o_ref[...] = f_ref[...] * norm


def _fin_body(a_ref, d_ref, w_ref, b_ref, o_ref):
    a = a_ref[0] + a_ref[1]
    d = d_ref[0] + d_ref[1]
    norm = lax.rsqrt(jnp.maximum(d, 1.0))
    y = jnp.dot(a * norm, w_ref[...], preferred_element_type=jnp.float32)
    o_ref[...] = jnp.maximum(y + b_ref[...], 0.0)


def kernel(features, edge_index, W, b):
    N, D = features.shape
    E = edge_index.shape[1]
    NP = ((N + _NW * _L - 1) // (_NW * _L)) * (_NW * _L)
    src = edge_index[0]
    dst = edge_index[1]

    degp = _make_deg_kernel(E, NP)(src, dst)
    deg_out = degp[:, 0, :N, None]
    deg_in = degp[:, 1, :N, None]

    R = 400
    assert N % R == 0
    h = pl.pallas_call(
        _h_body,
        grid=(N // R,),
        in_specs=[
            pl.BlockSpec((R, D), lambda i: (i, 0)),
            pl.BlockSpec((_NC, R, 1), lambda i: (0, i, 0)),
        ],
        out_specs=pl.BlockSpec((R, D), lambda i: (i, 0)),
        out_shape=jax.ShapeDtypeStruct((N, D), jnp.float32),
    )(features, deg_out)

    aggp = _make_agg_kernel(E, NP, D)(h, src, dst)

    out = pl.pallas_call(
        _fin_body,
        grid=(N // R,),
        in_specs=[
            pl.BlockSpec((_NC, R, D), lambda i: (0, i, 0)),
            pl.BlockSpec((_NC, R, 1), lambda i: (0, i, 0)),
            pl.BlockSpec((D, D), lambda i: (0, 0)),
            pl.BlockSpec((1, D), lambda i: (0, 0)),
        ],
        out_specs=pl.BlockSpec((R, D), lambda i: (i, 0)),
        out_shape=jax.ShapeDtypeStruct((N, D), jnp.float32),
    )(aggp[:, :N, :], deg_in, W, b.reshape(1, D))
    return out


# speedup vs baseline: 11.5968x; 2.4713x over previous
import functools

import jax
import jax.numpy as jnp
from jax import lax
from jax.experimental import pallas as pl
from jax.experimental.pallas import tpu as pltpu
from jax.experimental.pallas import tpu_sc as plsc

_NC = 2
_NS = 16
_NW = _NC * _NS
_L = 16
_C = 80


def _sc_mesh():
    return plsc.VectorSubcoreMesh(core_axis_name="c", subcore_axis_name="s")


@functools.lru_cache(maxsize=None)
def _make_deg_kernel(E: int, NP: int):
    EPT = E // _NW
    NCH = EPT // _C
    assert NCH * _C == EPT
    G = 5
    assert NCH % G == 0
    RPT = NP // _NS

    @functools.partial(
        pl.kernel,
        out_type=jax.ShapeDtypeStruct((_NC, NP), jnp.float32),
        mesh=_sc_mesh(),
        scratch_types=[
            pltpu.VMEM((NCH, _C), jnp.int32),
            pltpu.VMEM((_C,), jnp.float32),
            pltpu.VMEM((RPT,), jnp.float32),
            pltpu.VMEM_SHARED((NP,), jnp.float32),
            pltpu.SemaphoreType.DMA,
        ],
    )
    def deg_kernel(src_hbm, out_hbm, idx_all, ones_v, zero_v, deg_sh, sem):
        c = lax.axis_index("c")
        s = lax.axis_index("s")
        wid = c * _NS + s

        for j in range(_C // _L):
            ones_v[pl.ds(j * _L, _L)] = jnp.ones((_L,), jnp.float32)

        def zfill(i, carry):
            zero_v[pl.ds(i * _L, _L)] = jnp.zeros((_L,), jnp.float32)
            return carry
        lax.fori_loop(0, RPT // _L, zfill, 0)

        off = s * RPT
        pltpu.sync_copy(zero_v, deg_sh.at[pl.ds(off, RPT)])
        plsc.subcore_barrier()

        pltpu.sync_copy(src_hbm.at[wid], idx_all)

        def body(i, carry):
            descs = [
                pltpu.async_copy(ones_v, deg_sh.at[idx_all.at[i * G + g]],
                                 sem, add=True)
                for g in range(G)
            ]
            for d in descs:
                d.wait()
            return carry
        lax.fori_loop(0, NCH // G, body, 0)

        plsc.subcore_barrier()
        pltpu.sync_copy(deg_sh.at[pl.ds(off, RPT)],
                        out_hbm.at[c, pl.ds(off, RPT)])

    return deg_kernel


@functools.lru_cache(maxsize=None)
def _make_agg_kernel(E: int, NP: int, D: int):
    Dh = D // 2
    EPT = E // _NS
    NCH = EPT // _C
    assert NCH * _C == EPT
    NB = 4
    NG = NCH // NB
    RPT = NP // _NS
    ZR = 64

    @functools.partial(
        pl.kernel,
        out_type=(jax.ShapeDtypeStruct((_NC, NP, Dh), jnp.float32),
                  jax.ShapeDtypeStruct((NP,), jnp.float32)),
        mesh=_sc_mesh(),
        compiler_params=pltpu.CompilerParams(use_tc_tiling_on_sc=False),
        scratch_types=[
            pltpu.VMEM((NCH, _C), jnp.int32),
            pltpu.VMEM((NCH, _C), jnp.int32),
            pltpu.VMEM((_C, Dh), jnp.float32),
            pltpu.VMEM((_C, Dh), jnp.float32),
            pltpu.VMEM((_C, Dh), jnp.float32),
            pltpu.VMEM((_C, Dh), jnp.float32),
            pltpu.VMEM((ZR, Dh), jnp.float32),
            pltpu.VMEM((_C,), jnp.float32),
            pltpu.VMEM((RPT,), jnp.float32),
            pltpu.VMEM_SHARED((NP, Dh), jnp.float32),
            pltpu.VMEM_SHARED((NP,), jnp.float32),
            pltpu.SemaphoreType.DMA,
            pltpu.SemaphoreType.DMA,
            pltpu.SemaphoreType.DMA,
            pltpu.SemaphoreType.DMA,
            pltpu.SemaphoreType.DMA,
        ],
    )
    def agg_kernel(h_hbm, src_hbm, dst_hbm, agg_hbm, din_hbm,
                   src_all, dst_all, r0, r1, r2, r3, zbuf, ones_v, zero_v,
                   agg_sh, din_sh, sg0, sg1, sg2, sg3, sem_d):
        rows = (r0, r1, r2, r3)
        sems = (sg0, sg1, sg2, sg3)
        c = lax.axis_index("c")
        s = lax.axis_index("s")

        for j in range(_C // _L):
            ones_v[pl.ds(j * _L, _L)] = jnp.ones((_L,), jnp.float32)

        def zfill(i, carry):
            for j in range(Dh // _L):
                zbuf[i, pl.ds(j * _L, _L)] = jnp.zeros((_L,), jnp.float32)
            return carry
        lax.fori_loop(0, ZR, zfill, 0)

        def zfill1(i, carry):
            zero_v[pl.ds(i * _L, _L)] = jnp.zeros((_L,), jnp.float32)
            return carry
        lax.fori_loop(0, RPT // _L, zfill1, 0)

        def zcopy(k, carry):
            pltpu.sync_copy(zbuf, agg_sh.at[pl.ds(s * RPT + k * ZR, ZR), :])
            return carry
        lax.fori_loop(0, RPT // ZR, zcopy, 0)

        @pl.when(c == 0)
        def _():
            pltpu.sync_copy(zero_v, din_sh.at[pl.ds(s * RPT, RPT)])
        plsc.subcore_barrier()

        pltpu.sync_copy(src_hbm.at[s], src_all)
        pltpu.sync_copy(dst_hbm.at[s], dst_all)

        for b in range(NB):
            pltpu.async_copy(h_hbm.at[c].at[src_all.at[b]], rows[b], sems[b])

        def slot(k, b):
            pltpu.make_async_copy(h_hbm.at[c].at[src_all.at[k]], rows[b],
                                  sems[b]).wait()
            pltpu.sync_copy(rows[b], agg_sh.at[dst_all.at[k]], add=True)

            @pl.when(c == 0)
            def _():
                pltpu.async_copy(ones_v, din_sh.at[dst_all.at[k]], sem_d,
                                 add=True)

            @pl.when((c == 0) & (k >= NB))
            def _():
                pltpu.make_async_copy(ones_v, din_sh.at[dst_all.at[k]],
                                      sem_d).wait()

            @pl.when(k + NB < NCH)
            def _():
                pltpu.async_copy(h_hbm.at[c].at[src_all.at[k + NB]], rows[b],
                                 sems[b])

        def body(i, carry):
            for b in range(NB):
                slot(i * NB + b, b)
            return carry
        lax.fori_loop(0, NG, body, 0)
        for k in range(NG * NB, NCH):
            slot(k, k - NG * NB)

        @pl.when(c == 0)
        def _():
            for _i in range(NB):
                pltpu.make_async_copy(ones_v, din_sh.at[dst_all.at[0]],
                                      sem_d).wait()

        plsc.subcore_barrier()
        pltpu.sync_copy(agg_sh.at[pl.ds(s * RPT, RPT), :],
                        agg_hbm.at[c, pl.ds(s * RPT, RPT), :])

        @pl.when(c == 0)
        def _():
            pltpu.sync_copy(din_sh.at[pl.ds(s * RPT, RPT)],
                            din_hbm.at[pl.ds(s * RPT, RPT)])

    return agg_kernel


def _h_body(f_ref, d_ref, o_ref):
    d = d_ref[0] + d_ref[1]
    norm = lax.rsqrt(jnp.maximum(d, 1.0))
    x = f_ref[...] * norm
    Dh = x.shape[1] // 2
    o_ref[0] = x[:, :Dh]
    o_ref[1] = x[:, Dh:]


def _fin_body(a_ref, d_ref, w_ref, b_ref, o_ref):
    a = jnp.concatenate([a_ref[0], a_ref[1]], axis=1)
    norm = lax.rsqrt(jnp.maximum(d_ref[...], 1.0))
    y = jnp.dot(a * norm, w_ref[...], preferred_element_type=jnp.float32)
    o_ref[...] = jnp.maximum(y + b_ref[...], 0.0)


def kernel(features, edge_index, W, b):
    N, D = features.shape
    E = edge_index.shape[1]
    NP = ((N + _NW * _L - 1) // (_NW * _L)) * (_NW * _L)
    EPT = E // _NW
    src3d = edge_index[0].reshape(_NW, EPT // _C, _C)
    dst3d = edge_index[1].reshape(_NW, EPT // _C, _C)
    srcT = edge_index[0].reshape(_NS, (E // _NS) // _C, _C)
    dstT = edge_index[1].reshape(_NS, (E // _NS) // _C, _C)

    degp = _make_deg_kernel(E, NP)(src3d)
    deg_out = degp[:, :N, None]

    R = 400
    assert N % R == 0
    Dh = D // 2
    h2 = pl.pallas_call(
        _h_body,
        grid=(N // R,),
        in_specs=[
            pl.BlockSpec((R, D), lambda i: (i, 0)),
            pl.BlockSpec((_NC, R, 1), lambda i: (0, i, 0)),
        ],
        out_specs=pl.BlockSpec((_NC, R, Dh), lambda i: (0, i, 0)),
        out_shape=jax.ShapeDtypeStruct((_NC, N, Dh), jnp.float32),
    )(features, deg_out)

    aggp, din = _make_agg_kernel(E, NP, D)(h2, srcT, dstT)

    out = pl.pallas_call(
        _fin_body,
        grid=(N // R,),
        in_specs=[
            pl.BlockSpec((_NC, R, Dh), lambda i: (0, i, 0)),
            pl.BlockSpec((R, 1), lambda i: (i, 0)),
            pl.BlockSpec((D, D), lambda i: (0, 0)),
            pl.BlockSpec((1, D), lambda i: (0, 0)),
        ],
        out_specs=pl.BlockSpec((R, D), lambda i: (i, 0)),
        out_shape=jax.ShapeDtypeStruct((N, D), jnp.float32),
    )(aggp[:, :N, :], din[:N, None], W, b.reshape(1, D))
    return out


# speedup vs baseline: 14.2518x; 1.2289x over previous
import functools

import jax
import jax.numpy as jnp
from jax import lax
from jax.experimental import pallas as pl
from jax.experimental.pallas import tpu as pltpu
from jax.experimental.pallas import tpu_sc as plsc

_NC = 2
_NS = 16
_NW = _NC * _NS
_L = 16
_C = 80


def _sc_mesh():
    return plsc.VectorSubcoreMesh(core_axis_name="c", subcore_axis_name="s")


@functools.lru_cache(maxsize=None)
def _make_deg_kernel(E: int, NP: int):
    EPT = E // _NW
    NCH = EPT // _C
    assert NCH * _C == EPT
    G = 5
    assert NCH % G == 0
    RPT = NP // _NS

    @functools.partial(
        pl.kernel,
        out_type=jax.ShapeDtypeStruct((_NC, NP), jnp.float32),
        mesh=_sc_mesh(),
        compiler_params=pltpu.CompilerParams(use_tc_tiling_on_sc=False),
        scratch_types=[
            pltpu.VMEM((NCH, _C), jnp.int32),
            pltpu.VMEM((_C,), jnp.float32),
            pltpu.VMEM((RPT,), jnp.float32),
            pltpu.VMEM_SHARED((NP,), jnp.float32),
            pltpu.SemaphoreType.DMA,
        ],
    )
    def deg_kernel(ei_hbm, out_hbm, idx_all, ones_v, zero_v, deg_sh, sem):
        c = lax.axis_index("c")
        s = lax.axis_index("s")

        for j in range(_C // _L):
            ones_v[pl.ds(j * _L, _L)] = jnp.ones((_L,), jnp.float32)

        def zfill(i, carry):
            zero_v[pl.ds(i * _L, _L)] = jnp.zeros((_L,), jnp.float32)
            return carry
        lax.fori_loop(0, RPT // _L, zfill, 0)

        off = s * RPT
        pltpu.sync_copy(zero_v, deg_sh.at[pl.ds(off, RPT)])
        plsc.subcore_barrier()

        pltpu.sync_copy(ei_hbm.at[0, s, pl.ds(c * NCH, NCH), :], idx_all)

        def body(i, carry):
            descs = [
                pltpu.async_copy(ones_v, deg_sh.at[idx_all.at[i * G + g]],
                                 sem, add=True)
                for g in range(G)
            ]
            for d in descs:
                d.wait()
            return carry
        lax.fori_loop(0, NCH // G, body, 0)

        plsc.subcore_barrier()
        pltpu.sync_copy(deg_sh.at[pl.ds(off, RPT)],
                        out_hbm.at[c, pl.ds(off, RPT)])

    return deg_kernel


@functools.lru_cache(maxsize=None)
def _make_agg_kernel(E: int, NP: int, D: int):
    Dh = D // 2
    EPT = E // _NS
    NCH = EPT // _C
    assert NCH * _C == EPT
    NB = 4
    NG = NCH // NB
    RPT = NP // _NS
    ZR = 64

    @functools.partial(
        pl.kernel,
        out_type=(jax.ShapeDtypeStruct((_NC, NP, Dh), jnp.float32),
                  jax.ShapeDtypeStruct((NP,), jnp.float32)),
        mesh=_sc_mesh(),
        compiler_params=pltpu.CompilerParams(use_tc_tiling_on_sc=False),
        scratch_types=[
            pltpu.VMEM((NCH, _C), jnp.int32),
            pltpu.VMEM((NCH, _C), jnp.int32),
            pltpu.VMEM((_C, Dh), jnp.float32),
            pltpu.VMEM((_C, Dh), jnp.float32),
            pltpu.VMEM((_C, Dh), jnp.float32),
            pltpu.VMEM((_C, Dh), jnp.float32),
            pltpu.VMEM((ZR, Dh), jnp.float32),
            pltpu.VMEM((_C,), jnp.float32),
            pltpu.VMEM((RPT,), jnp.float32),
            pltpu.VMEM_SHARED((NP, Dh), jnp.float32),
            pltpu.VMEM_SHARED((NP,), jnp.float32),
            pltpu.SemaphoreType.DMA,
            pltpu.SemaphoreType.DMA,
            pltpu.SemaphoreType.DMA,
            pltpu.SemaphoreType.DMA,
            pltpu.SemaphoreType.DMA,
        ],
    )
    def agg_kernel(h_hbm, ei_hbm, agg_hbm, din_hbm,
                   src_all, dst_all, r0, r1, r2, r3, zbuf, ones_v, zero_v,
                   agg_sh, din_sh, sg0, sg1, sg2, sg3, sem_d):
        rows = (r0, r1, r2, r3)
        sems = (sg0, sg1, sg2, sg3)
        c = lax.axis_index("c")
        s = lax.axis_index("s")

        for j in range(_C // _L):
            ones_v[pl.ds(j * _L, _L)] = jnp.ones((_L,), jnp.float32)

        def zfill(i, carry):
            for j in range(Dh // _L):
                zbuf[i, pl.ds(j * _L, _L)] = jnp.zeros((_L,), jnp.float32)
            return carry
        lax.fori_loop(0, ZR, zfill, 0)

        def zfill1(i, carry):
            zero_v[pl.ds(i * _L, _L)] = jnp.zeros((_L,), jnp.float32)
            return carry
        lax.fori_loop(0, RPT // _L, zfill1, 0)

        def zcopy(k, carry):
            pltpu.sync_copy(zbuf, agg_sh.at[pl.ds(s * RPT + k * ZR, ZR), :])
            return carry
        lax.fori_loop(0, RPT // ZR, zcopy, 0)

        @pl.when(c == 0)
        def _():
            pltpu.sync_copy(zero_v, din_sh.at[pl.ds(s * RPT, RPT)])
        plsc.subcore_barrier()

        pltpu.sync_copy(ei_hbm.at[0, s], src_all)
        pltpu.sync_copy(ei_hbm.at[1, s], dst_all)

        for b in range(NB):
            pltpu.async_copy(h_hbm.at[c].at[src_all.at[b]], rows[b], sems[b])

        def slot(k, b):
            pltpu.make_async_copy(h_hbm.at[c].at[src_all.at[k]], rows[b],
                                  sems[b]).wait()
            pltpu.sync_copy(rows[b], agg_sh.at[dst_all.at[k]], add=True)

            @pl.when(c == 0)
            def _():
                pltpu.async_copy(ones_v, din_sh.at[dst_all.at[k]], sem_d,
                                 add=True)

            @pl.when((c == 0) & (k >= NB))
            def _():
                pltpu.make_async_copy(ones_v, din_sh.at[dst_all.at[k]],
                                      sem_d).wait()

            @pl.when(k + NB < NCH)
            def _():
                pltpu.async_copy(h_hbm.at[c].at[src_all.at[k + NB]], rows[b],
                                 sems[b])

        def body(i, carry):
            for b in range(NB):
                slot(i * NB + b, b)
            return carry
        lax.fori_loop(0, NG, body, 0)
        for k in range(NG * NB, NCH):
            slot(k, k - NG * NB)

        @pl.when(c == 0)
        def _():
            for _i in range(NB):
                pltpu.make_async_copy(ones_v, din_sh.at[dst_all.at[0]],
                                      sem_d).wait()

        plsc.subcore_barrier()
        pltpu.sync_copy(agg_sh.at[pl.ds(s * RPT, RPT), :],
                        agg_hbm.at[c, pl.ds(s * RPT, RPT), :])

        @pl.when(c == 0)
        def _():
            pltpu.sync_copy(din_sh.at[pl.ds(s * RPT, RPT)],
                            din_hbm.at[pl.ds(s * RPT, RPT)])

    return agg_kernel


def _h_body(f_ref, d_ref, o_ref):
    d = d_ref[0] + d_ref[1]
    norm = lax.rsqrt(jnp.maximum(d, 1.0))
    x = f_ref[...] * norm
    Dh = x.shape[1] // 2
    o_ref[0] = x[:, :Dh]
    o_ref[1] = x[:, Dh:]


def _fin_body(a_ref, d_ref, w_ref, b_ref, o_ref):
    a = jnp.concatenate([a_ref[0], a_ref[1]], axis=1)
    norm = lax.rsqrt(jnp.maximum(d_ref[...], 1.0))
    y = jnp.dot(a * norm, w_ref[...], preferred_element_type=jnp.float32)
    o_ref[...] = jnp.maximum(y + b_ref[...], 0.0)


def kernel(features, edge_index, W, b):
    N, D = features.shape
    E = edge_index.shape[1]
    NP = ((N + _NW * _L - 1) // (_NW * _L)) * (_NW * _L)
    EPS = E // _NS
    ei4 = edge_index.reshape(2, _NS, EPS // _C, _C)

    degp = _make_deg_kernel(E, NP)(ei4)
    deg_out = degp[:, :, None]

    R = 2000
    assert N % R == 0
    Dh = D // 2
    h2 = pl.pallas_call(
        _h_body,
        grid=(N // R,),
        in_specs=[
            pl.BlockSpec((R, D), lambda i: (i, 0)),
            pl.BlockSpec((_NC, R, 1), lambda i: (0, i, 0)),
        ],
        out_specs=pl.BlockSpec((_NC, R, Dh), lambda i: (0, i, 0)),
        out_shape=jax.ShapeDtypeStruct((_NC, N, Dh), jnp.float32),
    )(features, deg_out)

    aggp, din = _make_agg_kernel(E, NP, D)(h2, ei4)

    out = pl.pallas_call(
        _fin_body,
        grid=(N // R,),
        in_specs=[
            pl.BlockSpec((_NC, R, Dh), lambda i: (0, i, 0)),
            pl.BlockSpec((R, 1), lambda i: (i, 0)),
            pl.BlockSpec((D, D), lambda i: (0, 0)),
            pl.BlockSpec((1, D), lambda i: (0, 0)),
        ],
        out_specs=pl.BlockSpec((R, D), lambda i: (i, 0)),
        out_shape=jax.ShapeDtypeStruct((N, D), jnp.float32),
    )(aggp, din[:, None], W, b.reshape(1, D))
    return out


# speedup vs baseline: 15.4868x; 1.0866x over previous
import functools

import jax
import jax.numpy as jnp
from jax import lax
from jax.experimental import pallas as pl
from jax.experimental.pallas import tpu as pltpu
from jax.experimental.pallas import tpu_sc as plsc

_NC = 2
_NS = 16
_NW = _NC * _NS
_L = 16
_C = 80


def _sc_mesh():
    return plsc.VectorSubcoreMesh(core_axis_name="c", subcore_axis_name="s")


@functools.lru_cache(maxsize=None)
def _make_deg_kernel(E: int, NP: int):
    EPT = E // _NW
    NCH = EPT // _C
    assert NCH * _C == EPT
    G = 5
    assert NCH % G == 0
    RPT = NP // _NS

    @functools.partial(
        pl.kernel,
        out_type=jax.ShapeDtypeStruct((_NC, NP), jnp.float32),
        mesh=_sc_mesh(),
        compiler_params=pltpu.CompilerParams(use_tc_tiling_on_sc=False),
        scratch_types=[
            pltpu.VMEM((NCH, _C), jnp.int32),
            pltpu.VMEM((_C,), jnp.float32),
            pltpu.VMEM((RPT,), jnp.float32),
            pltpu.VMEM_SHARED((NP,), jnp.float32),
            pltpu.SemaphoreType.DMA,
        ],
    )
    def deg_kernel(ei_hbm, out_hbm, idx_all, ones_v, zero_v, deg_sh, sem):
        c = lax.axis_index("c")
        s = lax.axis_index("s")

        for j in range(_C // _L):
            ones_v[pl.ds(j * _L, _L)] = jnp.ones((_L,), jnp.float32)

        def zfill(i, carry):
            zero_v[pl.ds(i * _L, _L)] = jnp.zeros((_L,), jnp.float32)
            return carry
        lax.fori_loop(0, RPT // _L, zfill, 0)

        off = s * RPT
        pltpu.sync_copy(zero_v, deg_sh.at[pl.ds(off, RPT)])
        plsc.subcore_barrier()

        pltpu.sync_copy(ei_hbm.at[0, s, pl.ds(c * NCH, NCH), :], idx_all)

        def body(i, carry):
            descs = [
                pltpu.async_copy(ones_v, deg_sh.at[idx_all.at[i * G + g]],
                                 sem, add=True)
                for g in range(G)
            ]
            for d in descs:
                d.wait()
            return carry
        lax.fori_loop(0, NCH // G, body, 0)

        plsc.subcore_barrier()
        pltpu.sync_copy(deg_sh.at[pl.ds(off, RPT)],
                        out_hbm.at[c, pl.ds(off, RPT)])

    return deg_kernel


@functools.lru_cache(maxsize=None)
def _make_agg_kernel(E: int, NP: int, D: int):
    Dh = D // 2
    EPT = E // _NS
    NCH = EPT // _C
    assert NCH * _C == EPT
    NB = 4
    NG = NCH // NB
    RPT = NP // _NS
    ZR = 64

    @functools.partial(
        pl.kernel,
        out_type=(jax.ShapeDtypeStruct((_NC, NP, Dh), jnp.float32),
                  jax.ShapeDtypeStruct((NP,), jnp.float32)),
        mesh=_sc_mesh(),
        compiler_params=pltpu.CompilerParams(use_tc_tiling_on_sc=False),
        scratch_types=[
            pltpu.VMEM((NCH, _C), jnp.int32),
            pltpu.VMEM((NCH, _C), jnp.int32),
            pltpu.VMEM((_C, Dh), jnp.float32),
            pltpu.VMEM((_C, Dh), jnp.float32),
            pltpu.VMEM((_C, Dh), jnp.float32),
            pltpu.VMEM((_C, Dh), jnp.float32),
            pltpu.VMEM((ZR, Dh), jnp.float32),
            pltpu.VMEM((_C,), jnp.float32),
            pltpu.VMEM((RPT,), jnp.float32),
            pltpu.VMEM_SHARED((NP, Dh), jnp.float32),
            pltpu.VMEM_SHARED((NP,), jnp.float32),
            pltpu.SemaphoreType.DMA,
            pltpu.SemaphoreType.DMA,
            pltpu.SemaphoreType.DMA,
            pltpu.SemaphoreType.DMA,
            pltpu.SemaphoreType.DMA,
        ],
    )
    def agg_kernel(h_hbm, ei_hbm, agg_hbm, din_hbm,
                   src_all, dst_all, r0, r1, r2, r3, zbuf, ones_v, zero_v,
                   agg_sh, din_sh, sg0, sg1, sg2, sg3, sem_d):
        rows = (r0, r1, r2, r3)
        sems = (sg0, sg1, sg2, sg3)
        c = lax.axis_index("c")
        s = lax.axis_index("s")

        for j in range(_C // _L):
            ones_v[pl.ds(j * _L, _L)] = jnp.ones((_L,), jnp.float32)

        def zfill(i, carry):
            for j in range(Dh // _L):
                zbuf[i, pl.ds(j * _L, _L)] = jnp.zeros((_L,), jnp.float32)
            return carry
        lax.fori_loop(0, ZR, zfill, 0)

        def zfill1(i, carry):
            zero_v[pl.ds(i * _L, _L)] = jnp.zeros((_L,), jnp.float32)
            return carry
        lax.fori_loop(0, RPT // _L, zfill1, 0)

        def zcopy(k, carry):
            pltpu.sync_copy(zbuf, agg_sh.at[pl.ds(s * RPT + k * ZR, ZR), :])
            return carry
        lax.fori_loop(0, RPT // ZR, zcopy, 0)

        @pl.when(c == 0)
        def _():
            pltpu.sync_copy(zero_v, din_sh.at[pl.ds(s * RPT, RPT)])
        plsc.subcore_barrier()

        pltpu.sync_copy(ei_hbm.at[0, s], src_all)
        pltpu.sync_copy(ei_hbm.at[1, s], dst_all)

        for b in range(NB):
            pltpu.async_copy(h_hbm.at[c].at[src_all.at[b]], rows[b], sems[b])

        def slot(k, b):
            pltpu.make_async_copy(h_hbm.at[c].at[src_all.at[k]], rows[b],
                                  sems[b]).wait()
            pltpu.sync_copy(rows[b], agg_sh.at[dst_all.at[k]], add=True)

            @pl.when(c == 0)
            def _():
                pltpu.async_copy(ones_v, din_sh.at[dst_all.at[k]], sem_d,
                                 add=True)

            @pl.when((c == 0) & (k >= NB))
            def _():
                pltpu.make_async_copy(ones_v, din_sh.at[dst_all.at[k]],
                                      sem_d).wait()

            @pl.when(k + NB < NCH)
            def _():
                pltpu.async_copy(h_hbm.at[c].at[src_all.at[k + NB]], rows[b],
                                 sems[b])

        def body(i, carry):
            for b in range(NB):
                slot(i * NB + b, b)
            return carry
        lax.fori_loop(0, NG, body, 0)
        for k in range(NG * NB, NCH):
            slot(k, k - NG * NB)

        @pl.when(c == 0)
        def _():
            for _i in range(NB):
                pltpu.make_async_copy(ones_v, din_sh.at[dst_all.at[0]],
                                      sem_d).wait()

        plsc.subcore_barrier()
        pltpu.sync_copy(agg_sh.at[pl.ds(s * RPT, RPT), :],
                        agg_hbm.at[c, pl.ds(s * RPT, RPT), :])

        @pl.when(c == 0)
        def _():
            pltpu.sync_copy(din_sh.at[pl.ds(s * RPT, RPT)],
                            din_hbm.at[pl.ds(s * RPT, RPT)])

    return agg_kernel


def _h_body(f_ref, d_ref, o_ref):
    d = d_ref[0] + d_ref[1]
    norm = lax.rsqrt(jnp.maximum(d, 1.0))[:, None]
    x = f_ref[...] * norm
    Dh = x.shape[1] // 2
    o_ref[0] = x[:, :Dh]
    o_ref[1] = x[:, Dh:]


def _fin_body(a_ref, d_ref, w_ref, b_ref, o_ref):
    a = jnp.concatenate([a_ref[0], a_ref[1]], axis=1)
    norm = lax.rsqrt(jnp.maximum(d_ref[...], 1.0))[:, None]
    y = jnp.dot(a * norm, w_ref[...], preferred_element_type=jnp.float32)
    o_ref[...] = jnp.maximum(y + b_ref[...], 0.0)


def kernel(features, edge_index, W, b):
    N, D = features.shape
    E = edge_index.shape[1]
    NP = ((N + _NW * _L - 1) // (_NW * _L)) * (_NW * _L)
    EPS = E // _NS
    ei4 = edge_index.reshape(2, _NS, EPS // _C, _C)

    degp = _make_deg_kernel(E, NP)(ei4)

    R = 2048
    G = (N + R - 1) // R
    Dh = D // 2
    h2 = pl.pallas_call(
        _h_body,
        grid=(G,),
        in_specs=[
            pl.BlockSpec((R, D), lambda i: (i, 0)),
            pl.BlockSpec((_NC, R), lambda i: (0, i)),
        ],
        out_specs=pl.BlockSpec((_NC, R, Dh), lambda i: (0, i, 0)),
        out_shape=jax.ShapeDtypeStruct((_NC, N, Dh), jnp.float32),
    )(features, degp)

    aggp, din = _make_agg_kernel(E, NP, D)(h2, ei4)

    out = pl.pallas_call(
        _fin_body,
        grid=(G,),
        in_specs=[
            pl.BlockSpec((_NC, R, Dh), lambda i: (0, i, 0)),
            pl.BlockSpec((R,), lambda i: (i,)),
            pl.BlockSpec((D, D), lambda i: (0, 0)),
            pl.BlockSpec((1, D), lambda i: (0, 0)),
        ],
        out_specs=pl.BlockSpec((R, D), lambda i: (i, 0)),
        out_shape=jax.ShapeDtypeStruct((N, D), jnp.float32),
    )(aggp, din, W, b.reshape(1, D))
    return out


# speedup vs baseline: 15.7844x; 1.0192x over previous
import functools

import jax
import jax.numpy as jnp
from jax import lax
from jax.experimental import pallas as pl
from jax.experimental.pallas import tpu as pltpu
from jax.experimental.pallas import tpu_sc as plsc

_NC = 2
_NS = 16
_NW = _NC * _NS
_L = 16
_C = 80


def _sc_mesh():
    return plsc.VectorSubcoreMesh(core_axis_name="c", subcore_axis_name="s")


@functools.lru_cache(maxsize=None)
def _make_deg_kernel(E: int, NP: int):
    EPT = E // _NW
    NCH = EPT // _C
    assert NCH * _C == EPT
    G = 5
    assert NCH % G == 0
    RPT = NP // _NS

    @functools.partial(
        pl.kernel,
        out_type=jax.ShapeDtypeStruct((_NC, NP), jnp.float32),
        mesh=_sc_mesh(),
        compiler_params=pltpu.CompilerParams(use_tc_tiling_on_sc=False),
        scratch_types=[
            pltpu.VMEM((NCH, _C), jnp.int32),
            pltpu.VMEM((_C,), jnp.float32),
            pltpu.VMEM((RPT,), jnp.float32),
            pltpu.VMEM_SHARED((NP,), jnp.float32),
            pltpu.SemaphoreType.DMA,
        ],
    )
    def deg_kernel(ei_hbm, out_hbm, idx_all, ones_v, zero_v, deg_sh, sem):
        c = lax.axis_index("c")
        s = lax.axis_index("s")

        for j in range(_C // _L):
            ones_v[pl.ds(j * _L, _L)] = jnp.ones((_L,), jnp.float32)

        def zfill(i, carry):
            zero_v[pl.ds(i * _L, _L)] = jnp.zeros((_L,), jnp.float32)
            return carry
        lax.fori_loop(0, RPT // _L, zfill, 0)

        idx_dma = pltpu.async_copy(ei_hbm.at[0, s, pl.ds(c * NCH, NCH), :],
                                   idx_all, sem)
        off = s * RPT
        pltpu.sync_copy(zero_v, deg_sh.at[pl.ds(off, RPT)])
        idx_dma.wait()
        plsc.subcore_barrier()

        def body(i, carry):
            descs = [
                pltpu.async_copy(ones_v, deg_sh.at[idx_all.at[i * G + g]],
                                 sem, add=True)
                for g in range(G)
            ]
            for d in descs:
                d.wait()
            return carry
        lax.fori_loop(0, NCH // G, body, 0)

        plsc.subcore_barrier()
        pltpu.sync_copy(deg_sh.at[pl.ds(off, RPT)],
                        out_hbm.at[c, pl.ds(off, RPT)])

    return deg_kernel


@functools.lru_cache(maxsize=None)
def _make_agg_kernel(E: int, NP: int, D: int):
    Dh = D // 2
    EPT = E // _NS
    NCH = EPT // _C
    assert NCH * _C == EPT
    NB = 4
    NG = NCH // NB
    RPT = NP // _NS
    ZR = 64

    @functools.partial(
        pl.kernel,
        out_type=(jax.ShapeDtypeStruct((_NC, NP, Dh), jnp.float32),
                  jax.ShapeDtypeStruct((NP,), jnp.float32)),
        mesh=_sc_mesh(),
        compiler_params=pltpu.CompilerParams(use_tc_tiling_on_sc=False),
        scratch_types=[
            pltpu.VMEM((NCH, _C), jnp.int32),
            pltpu.VMEM((NCH, _C), jnp.int32),
            pltpu.VMEM((_C, Dh), jnp.float32),
            pltpu.VMEM((_C, Dh), jnp.float32),
            pltpu.VMEM((_C, Dh), jnp.float32),
            pltpu.VMEM((_C, Dh), jnp.float32),
            pltpu.VMEM((ZR, Dh), jnp.float32),
            pltpu.VMEM((_C,), jnp.float32),
            pltpu.VMEM((RPT,), jnp.float32),
            pltpu.VMEM_SHARED((NP, Dh), jnp.float32),
            pltpu.VMEM_SHARED((NP,), jnp.float32),
            pltpu.SemaphoreType.DMA,
            pltpu.SemaphoreType.DMA,
            pltpu.SemaphoreType.DMA,
            pltpu.SemaphoreType.DMA,
            pltpu.SemaphoreType.DMA,
        ],
    )
    def agg_kernel(h_hbm, ei_hbm, agg_hbm, din_hbm,
                   src_all, dst_all, r0, r1, r2, r3, zbuf, ones_v, zero_v,
                   agg_sh, din_sh, sg0, sg1, sg2, sg3, sem_d):
        rows = (r0, r1, r2, r3)
        sems = (sg0, sg1, sg2, sg3)
        c = lax.axis_index("c")
        s = lax.axis_index("s")

        src_dma = pltpu.async_copy(ei_hbm.at[0, s], src_all, sg0)
        dst_dma = pltpu.async_copy(ei_hbm.at[1, s], dst_all, sg1)

        for j in range(_C // _L):
            ones_v[pl.ds(j * _L, _L)] = jnp.ones((_L,), jnp.float32)

        def zfill(i, carry):
            for j in range(Dh // _L):
                zbuf[i, pl.ds(j * _L, _L)] = jnp.zeros((_L,), jnp.float32)
            return carry
        lax.fori_loop(0, ZR, zfill, 0)

        def zfill1(i, carry):
            zero_v[pl.ds(i * _L, _L)] = jnp.zeros((_L,), jnp.float32)
            return carry
        lax.fori_loop(0, RPT // _L, zfill1, 0)

        def zcopy(k, carry):
            pltpu.sync_copy(zbuf, agg_sh.at[pl.ds(s * RPT + k * ZR, ZR), :])
            return carry
        lax.fori_loop(0, RPT // ZR, zcopy, 0)

        @pl.when(c == 0)
        def _():
            pltpu.sync_copy(zero_v, din_sh.at[pl.ds(s * RPT, RPT)])
        src_dma.wait()
        dst_dma.wait()
        plsc.subcore_barrier()

        for b in range(NB):
            pltpu.async_copy(h_hbm.at[c].at[src_all.at[b]], rows[b], sems[b])

        def slot(k, b):
            pltpu.make_async_copy(h_hbm.at[c].at[src_all.at[k]], rows[b],
                                  sems[b]).wait()
            pltpu.sync_copy(rows[b], agg_sh.at[dst_all.at[k]], add=True)

            @pl.when(c == 0)
            def _():
                pltpu.async_copy(ones_v, din_sh.at[dst_all.at[k]], sem_d,
                                 add=True)

            @pl.when((c == 0) & (k >= NB))
            def _():
                pltpu.make_async_copy(ones_v, din_sh.at[dst_all.at[k]],
                                      sem_d).wait()

            @pl.when(k + NB < NCH)
            def _():
                pltpu.async_copy(h_hbm.at[c].at[src_all.at[k + NB]], rows[b],
                                 sems[b])

        def body(i, carry):
            for b in range(NB):
                slot(i * NB + b, b)
            return carry
        lax.fori_loop(0, NG, body, 0)
        for k in range(NG * NB, NCH):
            slot(k, k - NG * NB)

        @pl.when(c == 0)
        def _():
            for _i in range(NB):
                pltpu.make_async_copy(ones_v, din_sh.at[dst_all.at[0]],
                                      sem_d).wait()

        plsc.subcore_barrier()
        pltpu.sync_copy(agg_sh.at[pl.ds(s * RPT, RPT), :],
                        agg_hbm.at[c, pl.ds(s * RPT, RPT), :])

        @pl.when(c == 0)
        def _():
            pltpu.sync_copy(din_sh.at[pl.ds(s * RPT, RPT)],
                            din_hbm.at[pl.ds(s * RPT, RPT)])

    return agg_kernel


def _h_body(f_ref, d_ref, o_ref):
    d = d_ref[0] + d_ref[1]
    norm = lax.rsqrt(jnp.maximum(d, 1.0))[:, None]
    x = f_ref[...] * norm
    Dh = x.shape[1] // 2
    o_ref[0] = x[:, :Dh]
    o_ref[1] = x[:, Dh:]


def _fin_body(a_ref, d_ref, w_ref, b_ref, o_ref):
    a = jnp.concatenate([a_ref[0], a_ref[1]], axis=1)
    norm = lax.rsqrt(jnp.maximum(d_ref[...], 1.0))[:, None]
    y = jnp.dot(a * norm, w_ref[...], preferred_element_type=jnp.float32)
    o_ref[...] = jnp.maximum(y + b_ref[...], 0.0)


def kernel(features, edge_index, W, b):
    N, D = features.shape
    E = edge_index.shape[1]
    NP = ((N + _NW * _L - 1) // (_NW * _L)) * (_NW * _L)
    EPS = E // _NS
    ei4 = edge_index.reshape(2, _NS, EPS // _C, _C)

    degp = _make_deg_kernel(E, NP)(ei4)

    R = 2048
    G = (N + R - 1) // R
    Dh = D // 2
    h2 = pl.pallas_call(
        _h_body,
        grid=(G,),
        in_specs=[
            pl.BlockSpec((R, D), lambda i: (i, 0)),
            pl.BlockSpec((_NC, R), lambda i: (0, i)),
        ],
        out_specs=pl.BlockSpec((_NC, R, Dh), lambda i: (0, i, 0)),
        out_shape=jax.ShapeDtypeStruct((_NC, N, Dh), jnp.float32),
    )(features, degp)

    aggp, din = _make_agg_kernel(E, NP, D)(h2, ei4)

    out = pl.pallas_call(
        _fin_body,
        grid=(G,),
        in_specs=[
            pl.BlockSpec((_NC, R, Dh), lambda i: (0, i, 0)),
            pl.BlockSpec((R,), lambda i: (i,)),
            pl.BlockSpec((D, D), lambda i: (0, 0)),
            pl.BlockSpec((1, D), lambda i: (0, 0)),
        ],
        out_specs=pl.BlockSpec((R, D), lambda i: (i, 0)),
        out_shape=jax.ShapeDtypeStruct((N, D), jnp.float32),
    )(aggp, din, W, b.reshape(1, D))
    return out


# speedup vs baseline: 16.0291x; 1.0155x over previous
import functools

import jax
import jax.numpy as jnp
from jax import lax
from jax.experimental import pallas as pl
from jax.experimental.pallas import tpu as pltpu
from jax.experimental.pallas import tpu_sc as plsc

_NC = 2
_NS = 16
_NW = _NC * _NS
_L = 16
_C = 80


def _sc_mesh():
    return plsc.VectorSubcoreMesh(core_axis_name="c", subcore_axis_name="s")


@functools.lru_cache(maxsize=None)
def _make_deg_kernel(E: int, NP: int):
    EPT = E // _NW
    NCH = EPT // _C
    assert NCH * _C == EPT
    G = 5
    assert NCH % G == 0
    RPT = NP // _NS

    @functools.partial(
        pl.kernel,
        out_type=jax.ShapeDtypeStruct((_NC, NP), jnp.float32),
        mesh=_sc_mesh(),
        compiler_params=pltpu.CompilerParams(use_tc_tiling_on_sc=False),
        scratch_types=[
            pltpu.VMEM((NCH, _C), jnp.int32),
            pltpu.VMEM((_C,), jnp.float32),
            pltpu.VMEM((RPT,), jnp.float32),
            pltpu.VMEM_SHARED((NP,), jnp.float32),
            pltpu.SemaphoreType.DMA,
        ],
    )
    def deg_kernel(ei_hbm, out_hbm, idx_all, ones_v, zero_v, deg_sh, sem):
        c = lax.axis_index("c")
        s = lax.axis_index("s")

        for j in range(_C // _L):
            ones_v[pl.ds(j * _L, _L)] = jnp.ones((_L,), jnp.float32)

        def zfill(i, carry):
            zero_v[pl.ds(i * _L, _L)] = jnp.zeros((_L,), jnp.float32)
            return carry
        lax.fori_loop(0, RPT // _L, zfill, 0)

        idx_dma = pltpu.async_copy(ei_hbm.at[0, s, pl.ds(c * NCH, NCH), :],
                                   idx_all, sem)
        off = s * RPT
        pltpu.sync_copy(zero_v, deg_sh.at[pl.ds(off, RPT)])
        idx_dma.wait()
        plsc.subcore_barrier()

        def body(i, carry):
            descs = [
                pltpu.async_copy(ones_v, deg_sh.at[idx_all.at[i * G + g]],
                                 sem, add=True)
                for g in range(G)
            ]
            for d in descs:
                d.wait()
            return carry
        lax.fori_loop(0, NCH // G, body, 0)

        plsc.subcore_barrier()
        pltpu.sync_copy(deg_sh.at[pl.ds(off, RPT)],
                        out_hbm.at[c, pl.ds(off, RPT)])

    return deg_kernel


@functools.lru_cache(maxsize=None)
def _make_agg_kernel(E: int, NP: int, D: int):
    Dh = D // 2
    EPT = E // _NS
    NCH = EPT // _C
    assert NCH * _C == EPT
    NB = 4
    NG = NCH // NB
    RPT = NP // _NS
    ZR = 64

    @functools.partial(
        pl.kernel,
        out_type=(jax.ShapeDtypeStruct((_NC, NP, Dh), jnp.float32),
                  jax.ShapeDtypeStruct((NP,), jnp.float32)),
        mesh=_sc_mesh(),
        compiler_params=pltpu.CompilerParams(use_tc_tiling_on_sc=False),
        scratch_types=[
            pltpu.VMEM((NCH, _C), jnp.int32),
            pltpu.VMEM((NCH, _C), jnp.int32),
            pltpu.VMEM((_C, Dh), jnp.float32),
            pltpu.VMEM((_C, Dh), jnp.float32),
            pltpu.VMEM((_C, Dh), jnp.float32),
            pltpu.VMEM((_C, Dh), jnp.float32),
            pltpu.VMEM((ZR, Dh), jnp.float32),
            pltpu.VMEM((_C,), jnp.float32),
            pltpu.VMEM((RPT,), jnp.float32),
            pltpu.VMEM_SHARED((NP, Dh), jnp.float32),
            pltpu.VMEM_SHARED((NP,), jnp.float32),
            pltpu.SemaphoreType.DMA,
            pltpu.SemaphoreType.DMA,
            pltpu.SemaphoreType.DMA,
            pltpu.SemaphoreType.DMA,
            pltpu.SemaphoreType.DMA,
        ],
    )
    def agg_kernel(h_hbm, ei_hbm, agg_hbm, din_hbm,
                   src_all, dst_all, r0, r1, r2, r3, zbuf, ones_v, zero_v,
                   agg_sh, din_sh, sg0, sg1, sg2, sg3, sem_d):
        rows = (r0, r1, r2, r3)
        sems = (sg0, sg1, sg2, sg3)
        c = lax.axis_index("c")
        s = lax.axis_index("s")

        src_dma = pltpu.async_copy(ei_hbm.at[0, s], src_all, sg0)
        dst_dma = pltpu.async_copy(ei_hbm.at[1, s], dst_all, sg1)

        for j in range(_C // _L):
            ones_v[pl.ds(j * _L, _L)] = jnp.ones((_L,), jnp.float32)

        def zfill(i, carry):
            for j in range(Dh // _L):
                zbuf[i, pl.ds(j * _L, _L)] = jnp.zeros((_L,), jnp.float32)
            return carry
        lax.fori_loop(0, ZR, zfill, 0)

        def zfill1(i, carry):
            zero_v[pl.ds(i * _L, _L)] = jnp.zeros((_L,), jnp.float32)
            return carry
        lax.fori_loop(0, RPT // _L, zfill1, 0)

        def zcopy(k, carry):
            pltpu.sync_copy(zbuf, agg_sh.at[pl.ds(s * RPT + k * ZR, ZR), :])
            return carry
        lax.fori_loop(0, RPT // ZR, zcopy, 0)

        @pl.when(c == 0)
        def _():
            pltpu.sync_copy(zero_v, din_sh.at[pl.ds(s * RPT, RPT)])
        src_dma.wait()
        dst_dma.wait()
        plsc.subcore_barrier()

        for b in range(NB):
            pltpu.async_copy(h_hbm.at[c].at[src_all.at[b]], rows[b], sems[b])

        def slot(k, b):
            pltpu.make_async_copy(h_hbm.at[c].at[src_all.at[k]], rows[b],
                                  sems[b]).wait()
            pltpu.sync_copy(rows[b], agg_sh.at[dst_all.at[k]], add=True)

            @pl.when(c == 0)
            def _():
                pltpu.async_copy(ones_v, din_sh.at[dst_all.at[k]], sem_d,
                                 add=True)

            @pl.when((c == 0) & (k >= NB))
            def _():
                pltpu.make_async_copy(ones_v, din_sh.at[dst_all.at[k]],
                                      sem_d).wait()

            @pl.when(k + NB < NCH)
            def _():
                pltpu.async_copy(h_hbm.at[c].at[src_all.at[k + NB]], rows[b],
                                 sems[b])

        def body(i, carry):
            for b in range(NB):
                slot(i * NB + b, b)
            return carry
        lax.fori_loop(0, NG, body, 0)
        for k in range(NG * NB, NCH):
            slot(k, k - NG * NB)

        @pl.when(c == 0)
        def _():
            for _i in range(NB):
                pltpu.make_async_copy(ones_v, din_sh.at[dst_all.at[0]],
                                      sem_d).wait()

        plsc.subcore_barrier()
        pltpu.sync_copy(agg_sh.at[pl.ds(s * RPT, RPT), :],
                        agg_hbm.at[c, pl.ds(s * RPT, RPT), :])

        @pl.when(c == 0)
        def _():
            pltpu.sync_copy(din_sh.at[pl.ds(s * RPT, RPT)],
                            din_hbm.at[pl.ds(s * RPT, RPT)])

    return agg_kernel


def _h_body(f_ref, d_ref, o_ref):
    d = d_ref[0] + d_ref[1]
    norm = lax.rsqrt(jnp.maximum(d, 1.0))[:, None]
    x = f_ref[...] * norm
    Dh = x.shape[1] // 2
    o_ref[0] = x[:, :Dh]
    o_ref[1] = x[:, Dh:]


def _fin_body(a_ref, d_ref, w_ref, b_ref, o_ref):
    a = jnp.concatenate([a_ref[0], a_ref[1]], axis=1)
    norm = lax.rsqrt(jnp.maximum(d_ref[...], 1.0))[:, None]
    y = jnp.dot(a * norm, w_ref[...], preferred_element_type=jnp.float32)
    o_ref[...] = jnp.maximum(y + b_ref[...], 0.0)


def kernel(features, edge_index, W, b):
    N, D = features.shape
    E = edge_index.shape[1]
    NP = ((N + _NW * _L - 1) // (_NW * _L)) * (_NW * _L)
    EPS = E // _NS
    ei4 = edge_index.reshape(2, _NS, EPS // _C, _C)

    degp = _make_deg_kernel(E, NP)(ei4)

    R = 4096
    G = (N + R - 1) // R
    Dh = D // 2
    h2 = pl.pallas_call(
        _h_body,
        grid=(G,),
        in_specs=[
            pl.BlockSpec((R, D), lambda i: (i, 0)),
            pl.BlockSpec((_NC, R), lambda i: (0, i)),
        ],
        out_specs=pl.BlockSpec((_NC, R, Dh), lambda i: (0, i, 0)),
        out_shape=jax.ShapeDtypeStruct((_NC, N, Dh), jnp.float32),
    )(features, degp)

    aggp, din = _make_agg_kernel(E, NP, D)(h2, ei4)

    out = pl.pallas_call(
        _fin_body,
        grid=(G,),
        in_specs=[
            pl.BlockSpec((_NC, R, Dh), lambda i: (0, i, 0)),
            pl.BlockSpec((R,), lambda i: (i,)),
            pl.BlockSpec((D, D), lambda i: (0, 0)),
            pl.BlockSpec((1, D), lambda i: (0, 0)),
        ],
        out_specs=pl.BlockSpec((R, D), lambda i: (i, 0)),
        out_shape=jax.ShapeDtypeStruct((N, D), jnp.float32),
    )(aggp, din, W, b.reshape(1, D))
    return out
